# Initial kernel scaffold; baseline (speedup 1.0000x reference)
import jax
import jax.numpy as jnp
from jax.experimental import pallas as pl


def kernel(input_ids, weights):
    raise NotImplementedError("write your pallas kernel here")


# speedup vs baseline: 1.3349x; 1.3349x over previous
import functools

import jax
import jax.numpy as jnp
from jax import lax
from jax.experimental import pallas as pl
from jax.experimental.pallas import tpu as pltpu
from jax.experimental.pallas import tpu_sc as plsc

_B = 32
_S = 32
_V = 30522
_VPAD = 30528
_L = 16


def _row_body(ids_hbm, w_hbm, out_hbm, ids_v, g_v, row_v, sem):
    wid = lax.axis_index("s") * 2 + lax.axis_index("c")

    pltpu.sync_copy(ids_hbm.at[wid], ids_v)
    pltpu.async_copy(w_hbm.at[ids_v], g_v, sem).wait()

    def _zero(i, carry):
        row_v[pl.ds(i * _L, _L)] = jnp.zeros((_L,), jnp.float32)
        return carry

    lax.fori_loop(0, _VPAD // _L, _zero, 0)

    i0 = ids_v[pl.ds(0, _L)]
    i1 = ids_v[pl.ds(_L, _L)]
    g0 = g_v[pl.ds(0, _L)]
    g1 = g_v[pl.ds(_L, _L)]

    first = lax.broadcast(i0[0], (_L,))
    diff = (i0 ^ first) | (i1 ^ first)
    acc = diff[0]
    for j in range(1, _L):
        acc = acc | diff[j]
    eqv = lax.broadcast(acc == 0, (_L,))
    v0 = jnp.where(eqv, g0, jnp.maximum(g0, 0.0))
    v1 = jnp.where(eqv, g1, jnp.maximum(g1, 0.0))

    plsc.store_scatter(row_v, [i0], v0)
    plsc.store_scatter(row_v, [i1], v1)

    pltpu.sync_copy(row_v, out_hbm.at[wid])


@jax.jit
def _encode(input_ids, weights):
    call = functools.partial(
        pl.kernel,
        out_type=jax.ShapeDtypeStruct((_B, _VPAD), jnp.float32),
        mesh=plsc.VectorSubcoreMesh(core_axis_name="c", subcore_axis_name="s"),
        compiler_params=pltpu.CompilerParams(needs_layout_passes=False),
        scratch_types=[
            pltpu.VMEM((_S,), jnp.int32),
            pltpu.VMEM((_S,), jnp.float32),
            pltpu.VMEM((_VPAD,), jnp.float32),
            pltpu.SemaphoreType.DMA,
        ],
    )(_row_body)
    return call(input_ids, weights)


def kernel(input_ids, weights):
    out = _encode(input_ids, weights)
    return out[:, :_V].astype(jnp.bfloat16)


# speedup vs baseline: 1.5012x; 1.1246x over previous
import functools

import jax
import jax.numpy as jnp
from jax import lax
from jax.experimental import pallas as pl
from jax.experimental.pallas import tpu as pltpu
from jax.experimental.pallas import tpu_sc as plsc

_B = 32
_S = 32
_V = 30522
_VPAD = 30528
_L = 16


def _row_body(ids_hbm, w_hbm, out_hbm, ids_v, g_v, rowf, rowb, sem):
    wid = lax.axis_index("s") * 2 + lax.axis_index("c")

    pltpu.sync_copy(ids_hbm.at[wid], ids_v)
    gather = pltpu.async_copy(w_hbm.at[ids_v], g_v, sem)

    zero32 = jnp.zeros((2 * _L,), jnp.bfloat16)

    def _zero(i, carry):
        rowb[pl.ds(i * 2 * _L, 2 * _L)] = zero32
        return carry

    lax.fori_loop(0, _VPAD // (2 * _L), _zero, 0, unroll=8)

    gather.wait()

    i0 = ids_v[pl.ds(0, _L)]
    i1 = ids_v[pl.ds(_L, _L)]
    g0 = g_v[pl.ds(0, _L)]
    g1 = g_v[pl.ds(_L, _L)]

    first = lax.broadcast(i0[0], (_L,))
    diff = (i0 ^ first) | (i1 ^ first)
    acc = diff[0]
    for j in range(1, _L):
        acc = acc | diff[j]
    eqv = lax.broadcast(acc == 0, (_L,))
    v0 = jnp.where(eqv, g0, jnp.maximum(g0, 0.0))
    v1 = jnp.where(eqv, g1, jnp.maximum(g1, 0.0))

    def _xform(v):
        r = v & 31
        return (v & ~jnp.int32(31)) + (r >> 1) + ((v & 1) << 4)

    t0 = _xform(i0)
    t1 = _xform(i1)
    jv0 = i0 >> 5
    jv1 = i1 >> 5

    zero16 = jnp.zeros((_L,), jnp.float32)
    for s in range(_L):
        for jv in (jv0, jv1):
            base = jv[s] * 32
            rowf[pl.ds(base, _L)] = zero16
            rowf[pl.ds(base + _L, _L)] = zero16

    plsc.store_scatter(rowf, [t0], v0)
    plsc.store_scatter(rowf, [t1], v1)

    for s in range(_L):
        for jv in (jv0, jv1):
            base = jv[s] * 32
            a = rowf[pl.ds(base, _L)]
            b = rowf[pl.ds(base + _L, _L)]
            rowb[pl.ds(base, 2 * _L)] = plsc.pack(
                a, b, format=plsc.PackFormat.INTERLEAVED
            )

    pltpu.sync_copy(rowb, out_hbm.at[wid])


@jax.jit
def _encode(input_ids, weights):
    call = functools.partial(
        pl.kernel,
        out_type=jax.ShapeDtypeStruct((_B, _VPAD), jnp.bfloat16),
        mesh=plsc.VectorSubcoreMesh(core_axis_name="c", subcore_axis_name="s"),
        compiler_params=pltpu.CompilerParams(
            needs_layout_passes=False, use_tc_tiling_on_sc=False
        ),
        scratch_types=[
            pltpu.VMEM((_S,), jnp.int32),
            pltpu.VMEM((_S,), jnp.float32),
            pltpu.VMEM((_VPAD,), jnp.float32),
            pltpu.VMEM((_VPAD,), jnp.bfloat16),
            pltpu.SemaphoreType.DMA,
        ],
    )(_row_body)
    return call(input_ids, weights)


def kernel(input_ids, weights):
    return _encode(input_ids, weights)[:, :_V]


# speedup vs baseline: 1.7575x; 1.1707x over previous
import functools

import jax
import jax.numpy as jnp
from jax import lax
from jax.experimental import pallas as pl
from jax.experimental.pallas import tpu as pltpu
from jax.experimental.pallas import tpu_sc as plsc

_B = 32
_S = 32
_V = 30522
_NT = 239
_L = 16


def _row_body(ids_hbm, w_hbm, out_hbm, ids_v, g_v, rowf, sem):
    wid = lax.axis_index("s") * 2 + lax.axis_index("c")

    pltpu.sync_copy(ids_hbm.at[wid], ids_v)
    gather = pltpu.async_copy(w_hbm.at[ids_v], g_v, sem)

    zero16 = jnp.zeros((_L,), jnp.float32)

    def _zero(i, carry):
        rowf[i >> 3, pl.ds((i & 7) * _L, _L)] = zero16
        return carry

    lax.fori_loop(0, _NT * 8, _zero, 0, unroll=8)

    gather.wait()

    i0 = ids_v[pl.ds(0, _L)]
    i1 = ids_v[pl.ds(_L, _L)]
    g0 = g_v[pl.ds(0, _L)]
    g1 = g_v[pl.ds(_L, _L)]

    first = lax.broadcast(i0[0], (_L,))
    diff = (i0 ^ first) | (i1 ^ first)
    acc = diff[0]
    for j in range(1, _L):
        acc = acc | diff[j]
    eqv = lax.broadcast(acc == 0, (_L,))
    v0 = jnp.where(eqv, g0, jnp.maximum(g0, 0.0))
    v1 = jnp.where(eqv, g1, jnp.maximum(g1, 0.0))

    plsc.store_scatter(rowf, [i0 >> 7, i0 & 127], v0)
    plsc.store_scatter(rowf, [i1 >> 7, i1 & 127], v1)

    pltpu.sync_copy(rowf, out_hbm.at[wid >> 3, :, wid & 7, :])


@jax.jit
def _encode(input_ids, weights):
    call = functools.partial(
        pl.kernel,
        out_type=jax.ShapeDtypeStruct((_B // 8, _NT, 8, 128), jnp.float32),
        mesh=plsc.VectorSubcoreMesh(core_axis_name="c", subcore_axis_name="s"),
        compiler_params=pltpu.CompilerParams(
            needs_layout_passes=False, use_tc_tiling_on_sc=False
        ),
        scratch_types=[
            pltpu.VMEM((_S,), jnp.int32),
            pltpu.VMEM((_S,), jnp.float32),
            pltpu.VMEM((_NT, 128), jnp.float32),
            pltpu.SemaphoreType.DMA,
        ],
    )(_row_body)
    return call(input_ids, weights)


def kernel(input_ids, weights):
    tiles = _encode(input_ids, weights)
    full = jnp.transpose(tiles, (0, 2, 1, 3)).reshape(_B, _NT * 128)
    return full[:, :_V].astype(jnp.bfloat16)


# speedup vs baseline: 1.7693x; 1.0067x over previous
import functools

import jax
import jax.numpy as jnp
from jax import lax
from jax.experimental import pallas as pl
from jax.experimental.pallas import tpu as pltpu
from jax.experimental.pallas import tpu_sc as plsc

_B = 32
_S = 32
_V = 30522
_NT = 239
_L = 16


def _row_body(ids_hbm, w_hbm, out_hbm, ids_v, g_v, rowf, sem):
    wid = lax.axis_index("s") * 2 + lax.axis_index("c")

    pltpu.sync_copy(ids_hbm.at[wid], ids_v)
    gather = pltpu.async_copy(w_hbm.at[ids_v], g_v, sem)

    zero16 = jnp.zeros((_L,), jnp.float32)

    def _zero(j, carry):
        for c in range(8):
            rowf[j, pl.ds(c * _L, _L)] = zero16
        return carry

    lax.fori_loop(0, _NT, _zero, 0, unroll=2)

    gather.wait()

    i0 = ids_v[pl.ds(0, _L)]
    i1 = ids_v[pl.ds(_L, _L)]
    g0 = g_v[pl.ds(0, _L)]
    g1 = g_v[pl.ds(_L, _L)]

    first = lax.broadcast(i0[0], (_L,))
    diff = (i0 ^ first) | (i1 ^ first)
    acc = diff[0]
    for j in range(1, _L):
        acc = acc | diff[j]
    eqv = lax.broadcast(acc == 0, (_L,))
    v0 = jnp.where(eqv, g0, jnp.maximum(g0, 0.0))
    v1 = jnp.where(eqv, g1, jnp.maximum(g1, 0.0))

    plsc.store_scatter(rowf, [i0 >> 7, i0 & 127], v0)
    plsc.store_scatter(rowf, [i1 >> 7, i1 & 127], v1)

    pltpu.sync_copy(rowf, out_hbm.at[wid >> 3, :, wid & 7, :])


@jax.jit
def _encode(input_ids, weights):
    call = functools.partial(
        pl.kernel,
        out_type=jax.ShapeDtypeStruct((_B // 8, _NT, 8, 128), jnp.float32),
        mesh=plsc.VectorSubcoreMesh(core_axis_name="c", subcore_axis_name="s"),
        compiler_params=pltpu.CompilerParams(
            needs_layout_passes=False, use_tc_tiling_on_sc=False
        ),
        scratch_types=[
            pltpu.VMEM((_S,), jnp.int32),
            pltpu.VMEM((_S,), jnp.float32),
            pltpu.VMEM((_NT, 128), jnp.float32),
            pltpu.SemaphoreType.DMA,
        ],
    )(_row_body)
    return call(input_ids, weights)


def kernel(input_ids, weights):
    tiles = _encode(input_ids, weights)
    full = jnp.transpose(tiles, (0, 2, 1, 3)).reshape(_B, _NT * 128)
    return full[:, :_V].astype(jnp.bfloat16)
